# baseline (speedup 1.0000x reference)
import functools

import jax
import jax.numpy as jnp
from jax import lax
from jax.experimental import pallas as pl
from jax.experimental.pallas import tpu as pltpu
from jax.experimental.pallas import tpu_sc as plsc

B = 4096
L = 50
D = 64
NCLS = 6
V = 1000000
C8 = 8

NC = 2
NS = 16
NW = NC * NS
B_PER_W = B // NW
CHUNK = 128
NCHUNK = (B_PER_W * L) // CHUNK
ROWS_PER_SC = B // NC

BN = 4096


def _tw_body(wp_ref, tt_ref, twt_ref):
    twt_ref[...] = lax.dot_general(
        wp_ref[...].astype(jnp.bfloat16), tt_ref[...].astype(jnp.bfloat16),
        dimension_numbers=(((1,), (0,)), ((), ())),
        preferred_element_type=jnp.float32,
    ) * (1.0 / L)


def _tw(wp, table_t):
    return pl.pallas_call(
        _tw_body,
        grid=(pl.cdiv(V, BN),),
        in_specs=[
            pl.BlockSpec((C8, D), lambda i: (0, 0)),
            pl.BlockSpec((D, BN), lambda i: (0, i)),
        ],
        out_specs=pl.BlockSpec((C8, BN), lambda i: (0, i)),
        out_shape=jax.ShapeDtypeStruct((C8, V), jnp.float32),
    )(wp, table_t)


def _sc_pool_body(x_hbm, seg_hbm, tw_hbm, binit_hbm, out_hbm,
                  idxs, segs, rows, shared, gsem, ssem):
    c = lax.axis_index("c")
    s = lax.axis_index("s")
    wid = c * NS + s

    pltpu.sync_copy(x_hbm.at[wid], idxs)
    pltpu.sync_copy(seg_hbm.at[wid], segs)
    pltpu.sync_copy(binit_hbm, shared.at[pl.ds(s * B_PER_W, B_PER_W)])

    NBUF = rows.shape[0]
    for g in range(min(NBUF, NCHUNK)):
        pltpu.async_copy(tw_hbm.at[idxs.at[g]], rows.at[g % NBUF], gsem)
    for g in range(NCHUNK):
        pltpu.make_async_copy(
            tw_hbm.at[idxs.at[g]], rows.at[g % NBUF], gsem).wait()
        pltpu.async_copy(rows.at[g % NBUF], shared.at[segs.at[g]], ssem,
                         add=True)
        if g + NBUF < NCHUNK:
            pltpu.make_async_copy(
                rows.at[g % NBUF], shared.at[segs.at[g]], ssem).wait()
            pltpu.async_copy(tw_hbm.at[idxs.at[g + NBUF]],
                             rows.at[g % NBUF], gsem)
    for g in range(max(0, NCHUNK - NBUF), NCHUNK):
        pltpu.make_async_copy(
            rows.at[g % NBUF], shared.at[segs.at[g]], ssem).wait()

    pltpu.sync_copy(shared.at[pl.ds(s * B_PER_W, B_PER_W)],
                    out_hbm.at[pl.ds(wid * B_PER_W, B_PER_W)])


def _sc_pool(x_chunks, seg_chunks, tw, binit):
    mesh = plsc.VectorSubcoreMesh(core_axis_name="c", subcore_axis_name="s")
    kern = pl.kernel(
        _sc_pool_body,
        out_type=jax.ShapeDtypeStruct((B, C8), jnp.float32),
        mesh=mesh,
        scratch_types=[
            pltpu.VMEM((NCHUNK, CHUNK), jnp.int32),
            pltpu.VMEM((NCHUNK, CHUNK), jnp.int32),
            pltpu.VMEM((4, CHUNK, C8), jnp.float32),
            pltpu.VMEM_SHARED((ROWS_PER_SC, C8), jnp.float32),
            pltpu.SemaphoreType.DMA,
            pltpu.SemaphoreType.DMA,
        ],
        compiler_params=pltpu.CompilerParams(use_tc_tiling_on_sc=False),
    )
    return kern(x_chunks, seg_chunks, tw, binit)


def kernel(x, table, W, b):
    table_t = table.T
    wp = jnp.zeros((C8, D), jnp.float32).at[:NCLS].set(W)
    tw = _tw(wp, table_t).T

    x_chunks = x.astype(jnp.int32).reshape(NW, B_PER_W, L).transpose(0, 2, 1)
    seg_chunks = jnp.broadcast_to(
        ((jnp.arange(NW, dtype=jnp.int32)[:, None, None] % NS) * B_PER_W
         + jnp.arange(CHUNK, dtype=jnp.int32)[None, None, :]),
        (NW, NCHUNK, CHUNK))
    binit = jnp.zeros((B_PER_W, C8), jnp.float32).at[:, :NCLS].set(b)
    out8 = _sc_pool(x_chunks, seg_chunks, tw, binit)
    return out8[:, :NCLS]


# speedup vs baseline: 4.7968x; 4.7968x over previous
import jax
import jax.numpy as jnp
from jax import lax
from jax.experimental import pallas as pl
from jax.experimental.pallas import tpu as pltpu
from jax.experimental.pallas import tpu_sc as plsc

B = 4096
L = 50
D = 64
NCLS = 6
V = 1000000
C8 = 8

NC = 2
NS = 16
NW = NC * NS
B_PER_W = B // NW
CHUNK = 128
NCHUNK = (B_PER_W * L) // CHUNK
ROWS_PER_SC = B // NC

BN = 8192


def _tw_body(wp_ref, tt_ref, *plane_refs):
    res = lax.dot_general(
        wp_ref[...].astype(jnp.bfloat16), tt_ref[...].astype(jnp.bfloat16),
        dimension_numbers=(((1,), (0,)), ((), ())),
        preferred_element_type=jnp.float32,
    ) * (1.0 / L)
    for c in range(NCLS):
        plane_refs[c][...] = res[c, :]


def _tw_planes(wp, table_t):
    return pl.pallas_call(
        _tw_body,
        grid=(pl.cdiv(V, BN),),
        in_specs=[
            pl.BlockSpec((C8, D), lambda i: (0, 0)),
            pl.BlockSpec((D, BN), lambda i: (0, i)),
        ],
        out_specs=[pl.BlockSpec((BN,), lambda i: (i,))] * NCLS,
        out_shape=[jax.ShapeDtypeStruct((V,), jnp.float32)] * NCLS,
    )(wp, table_t)


def _sc_pool_body(x_hbm, seg8_hbm, binit_hbm,
                  p0, p1, p2, p3, p4, p5,
                  out_hbm, idxs, seg8, rows, acc, gsem, ssem):
    planes = (p0, p1, p2, p3, p4, p5)
    c = lax.axis_index("c")
    s = lax.axis_index("s")
    wid = c * NS + s

    pltpu.sync_copy(x_hbm.at[wid], idxs)
    pltpu.sync_copy(seg8_hbm.at[wid], seg8)
    pltpu.sync_copy(binit_hbm,
                    acc.at[pl.ds(s * B_PER_W * C8, B_PER_W * C8)])

    NBUF = rows.shape[0]
    for g in range(min(NBUF, NCHUNK)):
        for k in range(NCLS):
            pltpu.async_copy(planes[k].at[idxs.at[g]],
                             rows.at[g % NBUF, k], gsem)
    for g in range(NCHUNK):
        for k in range(NCLS):
            pltpu.make_async_copy(planes[k].at[idxs.at[g]],
                                  rows.at[g % NBUF, k], gsem).wait()
            pltpu.async_copy(rows.at[g % NBUF, k], acc.at[seg8.at[k]],
                             ssem, add=True)
        if g + NBUF < NCHUNK:
            for k in range(NCLS):
                pltpu.make_async_copy(rows.at[g % NBUF, k],
                                      acc.at[seg8.at[k]], ssem).wait()
                pltpu.async_copy(planes[k].at[idxs.at[g + NBUF]],
                                 rows.at[g % NBUF, k], gsem)
    for g in range(max(0, NCHUNK - NBUF), NCHUNK):
        for k in range(NCLS):
            pltpu.make_async_copy(rows.at[g % NBUF, k],
                                  acc.at[seg8.at[k]], ssem).wait()

    pltpu.sync_copy(acc.at[pl.ds(s * B_PER_W * C8, B_PER_W * C8)],
                    out_hbm.at[pl.ds(wid * B_PER_W * C8, B_PER_W * C8)])


def _sc_pool(x_chunks, seg8_chunks, binit, planes):
    mesh = plsc.VectorSubcoreMesh(core_axis_name="c", subcore_axis_name="s")
    kern = pl.kernel(
        _sc_pool_body,
        out_type=jax.ShapeDtypeStruct((B * C8,), jnp.float32),
        mesh=mesh,
        scratch_types=[
            pltpu.VMEM((NCHUNK, CHUNK), jnp.int32),
            pltpu.VMEM((NCLS, CHUNK), jnp.int32),
            pltpu.VMEM((4, NCLS, CHUNK), jnp.float32),
            pltpu.VMEM_SHARED((ROWS_PER_SC * C8,), jnp.float32),
            pltpu.SemaphoreType.DMA,
            pltpu.SemaphoreType.DMA,
        ],
        compiler_params=pltpu.CompilerParams(use_tc_tiling_on_sc=False),
    )
    return kern(x_chunks, seg8_chunks, binit, *planes)


def kernel(x, table, W, b):
    table_t = table.T
    wp = jnp.zeros((C8, D), jnp.float32).at[:NCLS].set(W)
    planes = _tw_planes(wp, table_t)

    x_chunks = x.astype(jnp.int32).reshape(NW, B_PER_W, L).transpose(0, 2, 1)
    seg8_chunks = ((jnp.arange(NW, dtype=jnp.int32)[:, None, None] % NS)
                   * (B_PER_W * C8)
                   + jnp.arange(CHUNK, dtype=jnp.int32)[None, None, :] * C8
                   + jnp.arange(NCLS, dtype=jnp.int32)[None, :, None])
    binit = jnp.broadcast_to(
        jnp.concatenate([b, jnp.zeros((C8 - NCLS,), jnp.float32)]),
        (B_PER_W, C8)).reshape(B_PER_W * C8)
    raw = _sc_pool(x_chunks, seg8_chunks, binit, planes)
    return raw.reshape(B, C8)[:, :NCLS]
